# baseline (speedup 1.0000x reference)
import functools

import jax
import jax.numpy as jnp
from jax import lax
from jax.experimental import pallas as pl
from jax.experimental.pallas import tpu as pltpu
from jax.experimental.pallas import tpu_sc as plsc

N = 10000
E = 320000
D = 128
H = 128
C = 20

NUM_SC = 2
TILES = 16
WORKERS = NUM_SC * TILES
EPT = E // WORKERS
KA = 80
KB = 64
GRP = 2 * KA + KB
NG = EPT // GRP
NPAD = 10240
RPT = NPAD // TILES
ZR = 80
WR = 128


def _sc_agg_body(x_hbm, ei_hbm, out_hbm, agg_sh, src_all, dst_all,
                 rows0, rows1, rows2, sem_i, g0, g1, g2, s0, s1, s2):
    c = lax.axis_index("c")
    s = lax.axis_index("s")
    w = c * TILES + s

    i0 = pltpu.async_copy(ei_hbm.at[pl.ds(w * EPT, EPT)], src_all, sem_i)
    i1 = pltpu.async_copy(ei_hbm.at[pl.ds(E + w * EPT, EPT)], dst_all, sem_i)

    def _gat(sem, rows, off, n):
        return pltpu.async_copy(
            x_hbm.at[src_all.at[pl.ds(off, n)]], rows.at[pl.ds(0, n)], sem)

    def _scat_async(sem, rows, off, n):
        return [
            pltpu.async_copy(
                rows.at[pl.ds(j * 16, 16)],
                agg_sh.at[dst_all[pl.ds(off + j * 16, 16)]],
                sem, add=True)
            for j in range(n // 16)
        ]

    i0.wait()
    i1.wait()
    _gat(g1, rows1, KA, KA)
    _gat(g2, rows2, 2 * KA, KB)

    def _zrow(r, _):
        def _zlane(j, _):
            rows0[r, pl.ds(j * 16, 16)] = jnp.zeros((16,), jnp.float32)
            return 0
        return lax.fori_loop(0, D // 16, _zlane, 0)
    lax.fori_loop(0, ZR, _zrow, 0)
    for b in range(RPT // ZR):
        pltpu.sync_copy(rows0, agg_sh.at[pl.ds(s * RPT + b * ZR, ZR)])
    plsc.subcore_barrier()
    _gat(g0, rows0, 0, KA)

    def _grp(g, _):
        base = g * GRP
        pltpu.make_async_copy(
            x_hbm.at[src_all.at[pl.ds(base, KA)]],
            rows0.at[pl.ds(0, KA)], g0).wait()
        ds0 = _scat_async(s0, rows0, base, KA)
        pltpu.make_async_copy(
            x_hbm.at[src_all.at[pl.ds(base + KA, KA)]],
            rows1.at[pl.ds(0, KA)], g1).wait()
        ds1 = _scat_async(s1, rows1, base + KA, KA)
        pltpu.make_async_copy(
            x_hbm.at[src_all.at[pl.ds(base + 2 * KA, KB)]],
            rows2.at[pl.ds(0, KB)], g2).wait()
        ds2 = _scat_async(s2, rows2, base + 2 * KA, KB)
        for d in ds0:
            d.wait()
        _gat(g0, rows0, base + GRP, KA)
        for d in ds1:
            d.wait()
        @pl.when(g < NG - 1)
        def _():
            _gat(g1, rows1, base + GRP + KA, KA)
        @pl.when(g == NG - 1)
        def _():
            _gat(g1, rows1, NG * GRP + KA, KB)
        for d in ds2:
            d.wait()
        @pl.when(g < NG - 1)
        def _():
            _gat(g2, rows2, base + GRP + 2 * KA, KB)
        return 0
    lax.fori_loop(0, NG, _grp, 0)

    tb = NG * GRP
    pltpu.make_async_copy(
        x_hbm.at[src_all.at[pl.ds(tb, KA)]], rows0.at[pl.ds(0, KA)], g0).wait()
    for d in _scat_async(s0, rows0, tb, KA):
        d.wait()
    pltpu.make_async_copy(
        x_hbm.at[src_all.at[pl.ds(tb + KA, KB)]],
        rows1.at[pl.ds(0, KB)], g1).wait()
    for d in _scat_async(s1, rows1, tb + KA, KB):
        d.wait()
    plsc.subcore_barrier()

    for b in range(RPT // WR):
        r0 = s * RPT + b * WR
        pltpu.sync_copy(agg_sh.at[pl.ds(r0, WR)], out_hbm.at[c, pl.ds(r0, WR)])


@functools.cache
def _sc_agg():
    return pl.kernel(
        _sc_agg_body,
        out_type=jax.ShapeDtypeStruct((NUM_SC, NPAD, D), jnp.float32),
        mesh=plsc.VectorSubcoreMesh(core_axis_name="c", subcore_axis_name="s"),
        scratch_types=[
            pltpu.VMEM_SHARED((NPAD, D), jnp.float32),
            pltpu.VMEM((EPT,), jnp.int32),
            pltpu.VMEM((EPT,), jnp.int32),
            pltpu.VMEM((KA, D), jnp.float32),
            pltpu.VMEM((KA, D), jnp.float32),
            pltpu.VMEM((KB, D), jnp.float32),
            pltpu.SemaphoreType.DMA,
            pltpu.SemaphoreType.DMA,
            pltpu.SemaphoreType.DMA,
            pltpu.SemaphoreType.DMA,
            pltpu.SemaphoreType.DMA,
            pltpu.SemaphoreType.DMA,
            pltpu.SemaphoreType.DMA,
        ],
    )


B = 2000
NB = N // B


def _head_body(x_ref, agg_ref, lab_ref, wc_ref, w1_ref, b1_ref, w2_ref, b2_ref,
               logp_ref, loss_ref, acc_ref):
    i = pl.program_id(0)
    z = x_ref[...] + agg_ref[0] + agg_ref[1]
    h = jnp.maximum(
        jnp.dot(z, w1_ref[...], preferred_element_type=jnp.float32) + b1_ref[...],
        0.0)
    logits = (jnp.dot(h, w2_ref[...], preferred_element_type=jnp.float32)
              + b2_ref[...])
    m = jnp.max(logits, axis=1, keepdims=True)
    lse = m + jnp.log(jnp.sum(jnp.exp(logits - m), axis=1, keepdims=True))
    logp = logits - lse
    logp_ref[...] = logp

    lab = lab_ref[0, 0, :]
    valid = (lab != -1)
    safe = jnp.where(valid, lab, 0)
    onehot = (safe[:, None] == lax.broadcasted_iota(jnp.int32, (B, C), 1))
    nll = -jnp.sum(jnp.where(onehot, logp, 0.0), axis=1)
    wrow = jnp.sum(jnp.where(onehot, wc_ref[...], 0.0), axis=1)
    wght = wrow * valid.astype(jnp.float32)

    @pl.when(i == 0)
    def _():
        acc_ref[0] = 0.0
        acc_ref[1] = 0.0
    acc_ref[0] += jnp.sum(wght * nll)
    acc_ref[1] += jnp.sum(wght)

    @pl.when(i == NB - 1)
    def _():
        loss_ref[...] = jnp.broadcast_to(acc_ref[0] / acc_ref[1], (1, 1))


def _head(x, aggp, labels3, wc2, W1, b12, W2, b22):
    return pl.pallas_call(
        _head_body,
        grid=(NB,),
        in_specs=[
            pl.BlockSpec((B, D), lambda i: (i, 0)),
            pl.BlockSpec((NUM_SC, B, D), lambda i: (0, i, 0)),
            pl.BlockSpec((1, 1, B), lambda i: (i, 0, 0)),
            pl.BlockSpec((1, C), lambda i: (0, 0)),
            pl.BlockSpec((D, H), lambda i: (0, 0)),
            pl.BlockSpec((1, H), lambda i: (0, 0)),
            pl.BlockSpec((H, C), lambda i: (0, 0)),
            pl.BlockSpec((1, C), lambda i: (0, 0)),
        ],
        out_specs=[
            pl.BlockSpec((B, C), lambda i: (i, 0)),
            pl.BlockSpec((1, 1), lambda i: (0, 0)),
        ],
        out_shape=[
            jax.ShapeDtypeStruct((N, C), jnp.float32),
            jax.ShapeDtypeStruct((1, 1), jnp.float32),
        ],
        scratch_shapes=[pltpu.SMEM((2,), jnp.float32)],
    )(x, aggp, labels3, wc2, W1, b12, W2, b22)


@jax.jit
def kernel(x, edge_index, labels, weight_classes, W1, b1, W2, b2):
    aggp = _sc_agg()(x, edge_index.reshape(2 * E))
    logp, loss = _head(
        x, aggp,
        labels.reshape(NB, 1, B),
        weight_classes.reshape(1, C),
        W1, b1.reshape(1, H), W2, b2.reshape(1, C),
    )
    return logp, loss.reshape(())


# speedup vs baseline: 1.1525x; 1.1525x over previous
import functools

import jax
import jax.numpy as jnp
from jax import lax
from jax.experimental import pallas as pl
from jax.experimental.pallas import tpu as pltpu
from jax.experimental.pallas import tpu_sc as plsc

N = 10000
E = 320000
D = 128
H = 128
C = 20

NUM_SC = 2
TILES = 16
WORKERS = NUM_SC * TILES
EPT = E // WORKERS
KA = 80
KB = 64
GRP = 2 * KA + KB
NG = EPT // GRP
NPAD = 10240
RPT = NPAD // TILES
ZR = 80
WR = 128


def _sc_agg_body(x_hbm, ei_hbm, out_hbm, agg_sh, src_all, dst_all,
                 rows0, rows1, rows2, sem_i, g0, g1, g2, s0, s1, s2):
    c = lax.axis_index("c")
    s = lax.axis_index("s")
    w = c * TILES + s

    i0 = pltpu.async_copy(ei_hbm.at[pl.ds(w * EPT, EPT)], src_all, sem_i)
    i1 = pltpu.async_copy(ei_hbm.at[pl.ds(E + w * EPT, EPT)], dst_all, sem_i)

    def _gat(sem, rows, off, n):
        return pltpu.async_copy(
            x_hbm.at[src_all.at[pl.ds(off, n)]], rows.at[pl.ds(0, n)], sem)

    def _scat_async(sem, rows, off, n):
        return [
            pltpu.async_copy(
                rows.at[pl.ds(j * 16, 16)],
                agg_sh.at[dst_all[pl.ds(off + j * 16, 16)]],
                sem, add=True)
            for j in range(n // 16)
        ]

    i0.wait()
    i1.wait()
    _gat(g1, rows1, KA, KA)
    _gat(g2, rows2, 2 * KA, KB)

    def _zrow(r, _):
        def _zlane(j, _):
            rows0[r, pl.ds(j * 16, 16)] = jnp.zeros((16,), jnp.float32)
            return 0
        return lax.fori_loop(0, D // 16, _zlane, 0)
    lax.fori_loop(0, ZR, _zrow, 0)
    for b in range(RPT // ZR):
        pltpu.sync_copy(rows0, agg_sh.at[pl.ds(s * RPT + b * ZR, ZR)])
    plsc.subcore_barrier()
    _gat(g0, rows0, 0, KA)

    def _grp(g, _):
        base = g * GRP
        pltpu.make_async_copy(
            x_hbm.at[src_all.at[pl.ds(base, KA)]],
            rows0.at[pl.ds(0, KA)], g0).wait()
        ds0 = _scat_async(s0, rows0, base, KA)
        for d in ds0:
            d.wait()
        _gat(g0, rows0, base + GRP, KA)
        pltpu.make_async_copy(
            x_hbm.at[src_all.at[pl.ds(base + KA, KA)]],
            rows1.at[pl.ds(0, KA)], g1).wait()
        ds1 = _scat_async(s1, rows1, base + KA, KA)
        for d in ds1:
            d.wait()
        @pl.when(g < NG - 1)
        def _():
            _gat(g1, rows1, base + GRP + KA, KA)
        @pl.when(g == NG - 1)
        def _():
            _gat(g1, rows1, NG * GRP + KA, KB)
        pltpu.make_async_copy(
            x_hbm.at[src_all.at[pl.ds(base + 2 * KA, KB)]],
            rows2.at[pl.ds(0, KB)], g2).wait()
        ds2 = _scat_async(s2, rows2, base + 2 * KA, KB)
        for d in ds2:
            d.wait()
        @pl.when(g < NG - 1)
        def _():
            _gat(g2, rows2, base + GRP + 2 * KA, KB)
        return 0
    lax.fori_loop(0, NG, _grp, 0)

    tb = NG * GRP
    pltpu.make_async_copy(
        x_hbm.at[src_all.at[pl.ds(tb, KA)]], rows0.at[pl.ds(0, KA)], g0).wait()
    for d in _scat_async(s0, rows0, tb, KA):
        d.wait()
    pltpu.make_async_copy(
        x_hbm.at[src_all.at[pl.ds(tb + KA, KB)]],
        rows1.at[pl.ds(0, KB)], g1).wait()
    for d in _scat_async(s1, rows1, tb + KA, KB):
        d.wait()
    plsc.subcore_barrier()

    for b in range(RPT // WR):
        r0 = s * RPT + b * WR
        pltpu.sync_copy(agg_sh.at[pl.ds(r0, WR)], out_hbm.at[c, pl.ds(r0, WR)])


@functools.cache
def _sc_agg():
    return pl.kernel(
        _sc_agg_body,
        out_type=jax.ShapeDtypeStruct((NUM_SC, NPAD, D), jnp.float32),
        mesh=plsc.VectorSubcoreMesh(core_axis_name="c", subcore_axis_name="s"),
        scratch_types=[
            pltpu.VMEM_SHARED((NPAD, D), jnp.float32),
            pltpu.VMEM((EPT,), jnp.int32),
            pltpu.VMEM((EPT,), jnp.int32),
            pltpu.VMEM((KA, D), jnp.float32),
            pltpu.VMEM((KA, D), jnp.float32),
            pltpu.VMEM((KB, D), jnp.float32),
            pltpu.SemaphoreType.DMA,
            pltpu.SemaphoreType.DMA,
            pltpu.SemaphoreType.DMA,
            pltpu.SemaphoreType.DMA,
            pltpu.SemaphoreType.DMA,
            pltpu.SemaphoreType.DMA,
            pltpu.SemaphoreType.DMA,
        ],
    )


B = 2000
NB = N // B


def _head_body(x_ref, agg_ref, lab_ref, wc_ref, w1_ref, b1_ref, w2_ref, b2_ref,
               logp_ref, loss_ref, acc_ref):
    i = pl.program_id(0)
    z = x_ref[...] + agg_ref[0] + agg_ref[1]
    h = jnp.maximum(
        jnp.dot(z, w1_ref[...], preferred_element_type=jnp.float32) + b1_ref[...],
        0.0)
    logits = (jnp.dot(h, w2_ref[...], preferred_element_type=jnp.float32)
              + b2_ref[...])
    m = jnp.max(logits, axis=1, keepdims=True)
    lse = m + jnp.log(jnp.sum(jnp.exp(logits - m), axis=1, keepdims=True))
    logp = logits - lse
    logp_ref[...] = logp

    lab = lab_ref[0, 0, :]
    valid = (lab != -1)
    safe = jnp.where(valid, lab, 0)
    onehot = (safe[:, None] == lax.broadcasted_iota(jnp.int32, (B, C), 1))
    nll = -jnp.sum(jnp.where(onehot, logp, 0.0), axis=1)
    wrow = jnp.sum(jnp.where(onehot, wc_ref[...], 0.0), axis=1)
    wght = wrow * valid.astype(jnp.float32)

    @pl.when(i == 0)
    def _():
        acc_ref[0] = 0.0
        acc_ref[1] = 0.0
    acc_ref[0] += jnp.sum(wght * nll)
    acc_ref[1] += jnp.sum(wght)

    @pl.when(i == NB - 1)
    def _():
        loss_ref[...] = jnp.broadcast_to(acc_ref[0] / acc_ref[1], (1, 1))


def _head(x, aggp, labels3, wc2, W1, b12, W2, b22):
    return pl.pallas_call(
        _head_body,
        grid=(NB,),
        in_specs=[
            pl.BlockSpec((B, D), lambda i: (i, 0)),
            pl.BlockSpec((NUM_SC, B, D), lambda i: (0, i, 0)),
            pl.BlockSpec((1, 1, B), lambda i: (i, 0, 0)),
            pl.BlockSpec((1, C), lambda i: (0, 0)),
            pl.BlockSpec((D, H), lambda i: (0, 0)),
            pl.BlockSpec((1, H), lambda i: (0, 0)),
            pl.BlockSpec((H, C), lambda i: (0, 0)),
            pl.BlockSpec((1, C), lambda i: (0, 0)),
        ],
        out_specs=[
            pl.BlockSpec((B, C), lambda i: (i, 0)),
            pl.BlockSpec((1, 1), lambda i: (0, 0)),
        ],
        out_shape=[
            jax.ShapeDtypeStruct((N, C), jnp.float32),
            jax.ShapeDtypeStruct((1, 1), jnp.float32),
        ],
        scratch_shapes=[pltpu.SMEM((2,), jnp.float32)],
    )(x, aggp, labels3, wc2, W1, b12, W2, b22)


@jax.jit
def kernel(x, edge_index, labels, weight_classes, W1, b1, W2, b2):
    aggp = _sc_agg()(x, edge_index.reshape(2 * E))
    logp, loss = _head(
        x, aggp,
        labels.reshape(NB, 1, B),
        weight_classes.reshape(1, C),
        W1, b1.reshape(1, H), W2, b2.reshape(1, C),
    )
    return logp, loss.reshape(())
